# Initial kernel scaffold; baseline (speedup 1.0000x reference)
import jax
import jax.numpy as jnp
from jax.experimental import pallas as pl


def kernel(x_series, Wp, bp, pos_enc, Wqkv, bqkv, Wo, bo, ln1_g, ln1_b, ln2_g, ln2_b, W1, b1, W2, b2, fin_g, fin_b, gate_W, gate_b, expW1, expb1, expW2, expb2, moe_g, moe_b, Wd, bd):
    raise NotImplementedError("write your pallas kernel here")


# speedup vs baseline: 1.9898x; 1.9898x over previous
import functools
import math

import jax
import jax.numpy as jnp
from jax.experimental import pallas as pl
from jax.experimental.pallas import tpu as pltpu

F32 = jnp.float32


def _ln(x, g, b):
    m = jnp.mean(x, axis=-1, keepdims=True)
    v = jnp.mean((x - m) ** 2, axis=-1, keepdims=True)
    return (x - m) * jax.lax.rsqrt(v + 1e-5) * g + b


def _gelu(x):
    return 0.5 * x * (1.0 + jax.lax.erf(x * (1.0 / math.sqrt(2.0))))


def _dot(a, b):
    return jnp.dot(a, b, preferred_element_type=F32)


def _decomp_proj_body(x_ref, wp_ref, bp_ref, pe_ref, o_ref):
    x = x_ref[0]
    t_, f_ = x.shape
    acc = x
    for off in (1, 2, 3):
        zpad = jnp.zeros((off, f_), x.dtype)
        acc = acc + jnp.concatenate([zpad, x[: t_ - off]], axis=0)
        acc = acc + jnp.concatenate([x[off:], zpad], axis=0)
    trend = acc * (1.0 / 7.0)
    resid = x - trend
    o_ref[0] = _dot(resid, wp_ref[...]) + bp_ref[...] + pe_ref[...]


def _mm_bias_body(x_ref, w_ref, b_ref, o_ref, act):
    y = _dot(x_ref[...], w_ref[...]) + b_ref[...]
    if act == "gelu":
        y = _gelu(y)
    o_ref[...] = y


def _proj_res_ln_body(x_ref, w_ref, b_ref, z_ref, g_ref, gb_ref, o_ref):
    t = z_ref[...] + _dot(x_ref[...], w_ref[...]) + b_ref[...]
    o_ref[...] = _ln(t, g_ref[...], gb_ref[...])


def _mm_bias(x, wt, b, act=None, bm=256):
    m, k = x.shape
    n = wt.shape[1]
    return pl.pallas_call(
        functools.partial(_mm_bias_body, act=act),
        grid=(m // bm,),
        in_specs=[
            pl.BlockSpec((bm, k), lambda i: (i, 0)),
            pl.BlockSpec((k, n), lambda i: (0, 0)),
            pl.BlockSpec((1, n), lambda i: (0, 0)),
        ],
        out_specs=pl.BlockSpec((bm, n), lambda i: (i, 0)),
        out_shape=jax.ShapeDtypeStruct((m, n), F32),
    )(x, wt, b.reshape(1, n))


def _proj_res_ln(x, wt, b, z, g, gb, bm=256):
    m, k = x.shape
    n = wt.shape[1]
    return pl.pallas_call(
        _proj_res_ln_body,
        grid=(m // bm,),
        in_specs=[
            pl.BlockSpec((bm, k), lambda i: (i, 0)),
            pl.BlockSpec((k, n), lambda i: (0, 0)),
            pl.BlockSpec((1, n), lambda i: (0, 0)),
            pl.BlockSpec((bm, n), lambda i: (i, 0)),
            pl.BlockSpec((1, n), lambda i: (0, 0)),
            pl.BlockSpec((1, n), lambda i: (0, 0)),
        ],
        out_specs=pl.BlockSpec((bm, n), lambda i: (i, 0)),
        out_shape=jax.ShapeDtypeStruct((m, n), F32),
    )(x, wt, b.reshape(1, n), z, g.reshape(1, n), gb.reshape(1, n))


def _softmax_ctx(q, k, v, scale):
    s = jax.lax.dot_general(q, k, (((1,), (1,)), ((), ())),
                            preferred_element_type=F32) * scale
    m = jnp.max(s, axis=-1, keepdims=True)
    e = jnp.exp(s - m)
    p = e / jnp.sum(e, axis=-1, keepdims=True)
    return _dot(p, v)


def _attn_full_body(q_ref, k_ref, v_ref, o_ref, scale, dh):
    q2 = q_ref[...]
    k2 = k_ref[...]
    v2 = v_ref[...]
    outs = []
    for i in range(2):
        sl = slice(i * dh, (i + 1) * dh)
        outs.append(_softmax_ctx(q2[:, sl], k2[:, sl], v2[:, sl], scale))
    o_ref[...] = jnp.concatenate(outs, axis=1)


def _attn_full(qkv, b, t, h, dh, d):
    scale = 1.0 / math.sqrt(float(dh))
    hp = h // 2
    return pl.pallas_call(
        functools.partial(_attn_full_body, scale=scale, dh=dh),
        grid=(b, hp),
        in_specs=[
            pl.BlockSpec((t, 2 * dh), lambda i, j: (i, j)),
            pl.BlockSpec((t, 2 * dh), lambda i, j, _hp=hp: (i, _hp + j)),
            pl.BlockSpec((t, 2 * dh), lambda i, j, _hp=hp: (i, 2 * _hp + j)),
        ],
        out_specs=pl.BlockSpec((t, 2 * dh), lambda i, j: (i, j)),
        out_shape=jax.ShapeDtypeStruct((b * t, d), F32),
    )(qkv, qkv, qkv)


def _attn_last_body(zl_ref, wq_ref, bq_ref, k_ref, v_ref, o_ref, scale, dh):
    zl = zl_ref[0]
    q2 = _dot(zl, wq_ref[...]) + bq_ref[...]
    k2 = k_ref[...]
    v2 = v_ref[...]
    outs = []
    for i in range(2):
        sl = slice(i * dh, (i + 1) * dh)
        outs.append(_softmax_ctx(q2[:, sl], k2[:, sl], v2[:, sl], scale))
    o_ref[0] = jnp.concatenate(outs, axis=1)


def _attn_last(z_last3, wqt, bq, kv, b, t, h, dh, d):
    scale = 1.0 / math.sqrt(float(dh))
    hp = h // 2
    return pl.pallas_call(
        functools.partial(_attn_last_body, scale=scale, dh=dh),
        grid=(b, hp),
        in_specs=[
            pl.BlockSpec((1, 1, d), lambda i, j: (i, 0, 0)),
            pl.BlockSpec((d, 2 * dh), lambda i, j: (0, j)),
            pl.BlockSpec((1, 2 * dh), lambda i, j: (0, j)),
            pl.BlockSpec((t, 2 * dh), lambda i, j: (i, j)),
            pl.BlockSpec((t, 2 * dh), lambda i, j, _hp=hp: (i, _hp + j)),
        ],
        out_specs=pl.BlockSpec((1, 1, 2 * dh), lambda i, j: (i, 0, j)),
        out_shape=jax.ShapeDtypeStruct((b, 1, d), F32),
    )(z_last3, wqt, bq.reshape(1, d), kv, kv)


def _tail_body(zl_ref, ctx_ref, wo_ref, bo_ref, g1_ref, b1n_ref,
               w1_ref, bf1_ref, w2_ref, bf2_ref, g2_ref, b2n_ref,
               fg_ref, fb_ref, gw_ref, gb_ref,
               ew1_ref, eb1_ref, ew2_ref, eb2_ref,
               mg_ref, mb_ref, wd_ref, bd_ref, xl_ref, o_ref, n_exp):
    zl = zl_ref[...]
    ctx = ctx_ref[...]
    z1 = _ln(zl + _dot(ctx, wo_ref[...]) + bo_ref[...],
             g1_ref[...], b1n_ref[...])
    ffh = _gelu(_dot(z1, w1_ref[...]) + bf1_ref[...])
    z2 = _ln(z1 + _dot(ffh, w2_ref[...]) + bf2_ref[...],
             g2_ref[...], b2n_ref[...])
    zf = _ln(z2, fg_ref[...], fb_ref[...])

    logits = _dot(zf, gw_ref[...]) + gb_ref[...]
    lcols = [logits[:, e:e + 1] for e in range(n_exp)]
    m1 = lcols[0]
    for le in lcols[1:]:
        m1 = jnp.maximum(m1, le)
    first, taken = [], None
    for le in lcols:
        is_e = (le == m1) if taken is None else jnp.logical_and(
            le == m1, jnp.logical_not(taken))
        first.append(is_e)
        taken = is_e if taken is None else jnp.logical_or(taken, is_e)
    masked = [jnp.where(f, -1e30, le) for f, le in zip(first, lcols)]
    m2 = masked[0]
    for le in masked[1:]:
        m2 = jnp.maximum(m2, le)
    second, taken2 = [], None
    for le in masked:
        is_e = (le == m2) if taken2 is None else jnp.logical_and(
            le == m2, jnp.logical_not(taken2))
        second.append(is_e)
        taken2 = is_e if taken2 is None else jnp.logical_or(taken2, is_e)
    w1c = 1.0 / (1.0 + jnp.exp(m2 - m1))
    mixed = jnp.zeros_like(zf)
    for e in range(n_exp):
        he = jnp.maximum(_dot(zf, ew1_ref[e]) + eb1_ref[e], 0.0)
        oe = _dot(he, ew2_ref[e]) + eb2_ref[e]
        coeff_e = w1c * first[e].astype(F32) + (1.0 - w1c) * second[e].astype(F32)
        mixed = mixed + coeff_e * oe

    zm = _ln(mixed + zf, mg_ref[...], mb_ref[...])
    xl = xl_ref[...]
    trend_last = (xl[:, 0] + xl[:, 1] + xl[:, 2] + xl[:, 3]) * (1.0 / 7.0)
    nr = wd_ref.shape[1]
    o_ref[...] = _dot(zm, wd_ref[...]) + bd_ref[...] + trend_last[:, :nr]


def kernel(x_series, Wp, bp, pos_enc, Wqkv, bqkv, Wo, bo, ln1_g, ln1_b,
           ln2_g, ln2_b, W1, b1, W2, b2, fin_g, fin_b, gate_W, gate_b,
           expW1, expb1, expW2, expb2, moe_g, moe_b, Wd, bd):
    b_, t_, f_ = x_series.shape
    d = Wp.shape[0]
    h = 8
    dh = d // h
    n_exp = gate_W.shape[0]
    nr = Wd.shape[0]
    n = b_ * t_

    z = pl.pallas_call(
        _decomp_proj_body,
        grid=(b_,),
        in_specs=[
            pl.BlockSpec((1, t_, f_), lambda i: (i, 0, 0)),
            pl.BlockSpec((f_, d), lambda i: (0, 0)),
            pl.BlockSpec((1, d), lambda i: (0, 0)),
            pl.BlockSpec((t_, d), lambda i: (0, 0)),
        ],
        out_specs=pl.BlockSpec((1, t_, d), lambda i: (i, 0, 0)),
        out_shape=jax.ShapeDtypeStruct((b_, t_, d), F32),
    )(x_series, Wp.T, bp.reshape(1, d), pos_enc[:t_])
    z = z.reshape(n, d)

    qkv = _mm_bias(z, Wqkv[0].T, bqkv[0])
    ctx = _attn_full(qkv, b_, t_, h, dh, d)
    z = _proj_res_ln(ctx, Wo[0].T, bo[0], z, ln1_g[0], ln1_b[0])
    ffh = _mm_bias(z, W1[0].T, b1[0], act="gelu")
    z = _proj_res_ln(ffh, W2[0].T, b2[0], z, ln2_g[0], ln2_b[0])

    kv = _mm_bias(z, Wqkv[1, d:].T, bqkv[1, d:])
    z_last = z.reshape(b_, t_, d)[:, -1, :]
    ctx_last = _attn_last(z_last.reshape(b_, 1, d), Wqkv[1, :d].T,
                          bqkv[1, :d], kv, b_, t_, h, dh, d)
    ctx_last = ctx_last.reshape(b_, d)

    def fullspec(shape):
        nd = len(shape)
        return pl.BlockSpec(shape, lambda *a, _nd=nd: (0,) * _nd)

    x_last4 = x_series[:, t_ - 4:, :]
    args = (
        z_last, ctx_last, Wo[1].T, bo[1].reshape(1, d),
        ln1_g[1].reshape(1, d), ln1_b[1].reshape(1, d),
        W1[1].T, b1[1].reshape(1, 4 * d), W2[1].T, b2[1].reshape(1, d),
        ln2_g[1].reshape(1, d), ln2_b[1].reshape(1, d),
        fin_g.reshape(1, d), fin_b.reshape(1, d),
        gate_W.T, gate_b.reshape(1, n_exp),
        expW1.transpose(0, 2, 1), expb1.reshape(n_exp, 1, 2 * d),
        expW2.transpose(0, 2, 1), expb2.reshape(n_exp, 1, d),
        moe_g.reshape(1, d), moe_b.reshape(1, d),
        Wd.T, bd.reshape(1, nr), x_last4,
    )
    out = pl.pallas_call(
        functools.partial(_tail_body, n_exp=n_exp),
        in_specs=[fullspec(a.shape) for a in args],
        out_specs=fullspec((b_, nr)),
        out_shape=jax.ShapeDtypeStruct((b_, nr), F32),
        compiler_params=pltpu.CompilerParams(
            vmem_limit_bytes=60 * 1024 * 1024),
    )(*args)
    return out


# speedup vs baseline: 2.5517x; 1.2824x over previous
import functools
import math

import jax
import jax.numpy as jnp
from jax.experimental import pallas as pl
from jax.experimental.pallas import tpu as pltpu

F32 = jnp.float32
BF16 = jnp.bfloat16


def _ln(x, g, b):
    m = jnp.mean(x, axis=-1, keepdims=True)
    v = jnp.mean((x - m) ** 2, axis=-1, keepdims=True)
    return (x - m) * jax.lax.rsqrt(v + 1e-5) * g + b


def _gelu(x):
    return 0.5 * x * (1.0 + jax.lax.erf(x * (1.0 / math.sqrt(2.0))))


def _dot(a, b):
    return jnp.dot(a.astype(BF16), b.astype(BF16),
                   preferred_element_type=F32)


def _decomp_proj_body(x_ref, wp_ref, bp_ref, pe_ref, o_ref):
    x = x_ref[0]
    t_, f_ = x.shape
    acc = x
    for off in (1, 2, 3):
        zpad = jnp.zeros((off, f_), x.dtype)
        acc = acc + jnp.concatenate([zpad, x[: t_ - off]], axis=0)
        acc = acc + jnp.concatenate([x[off:], zpad], axis=0)
    trend = acc * (1.0 / 7.0)
    resid = x - trend
    o_ref[0] = _dot(resid, wp_ref[...]) + bp_ref[...] + pe_ref[...]


def _mm_bias_body(x_ref, w_ref, b_ref, o_ref, act):
    y = _dot(x_ref[...], w_ref[...]) + b_ref[...]
    if act == "gelu":
        y = _gelu(y)
    o_ref[...] = y.astype(o_ref.dtype)


def _proj_res_ln_body(x_ref, w_ref, b_ref, z_ref, g_ref, gb_ref, o_ref):
    t = z_ref[...] + _dot(x_ref[...], w_ref[...]) + b_ref[...]
    o_ref[...] = _ln(t, g_ref[...], gb_ref[...])


def _mm_bias(x, wt, b, act=None, bm=256, out_dtype=F32):
    m, k = x.shape
    n = wt.shape[1]
    return pl.pallas_call(
        functools.partial(_mm_bias_body, act=act),
        grid=(m // bm,),
        in_specs=[
            pl.BlockSpec((bm, k), lambda i: (i, 0)),
            pl.BlockSpec((k, n), lambda i: (0, 0)),
            pl.BlockSpec((1, n), lambda i: (0, 0)),
        ],
        out_specs=pl.BlockSpec((bm, n), lambda i: (i, 0)),
        out_shape=jax.ShapeDtypeStruct((m, n), out_dtype),
    )(x, wt.astype(BF16), b.reshape(1, n))


def _proj_res_ln(x, wt, b, z, g, gb, bm=256):
    m, k = x.shape
    n = wt.shape[1]
    return pl.pallas_call(
        _proj_res_ln_body,
        grid=(m // bm,),
        in_specs=[
            pl.BlockSpec((bm, k), lambda i: (i, 0)),
            pl.BlockSpec((k, n), lambda i: (0, 0)),
            pl.BlockSpec((1, n), lambda i: (0, 0)),
            pl.BlockSpec((bm, n), lambda i: (i, 0)),
            pl.BlockSpec((1, n), lambda i: (0, 0)),
            pl.BlockSpec((1, n), lambda i: (0, 0)),
        ],
        out_specs=pl.BlockSpec((bm, n), lambda i: (i, 0)),
        out_shape=jax.ShapeDtypeStruct((m, n), F32),
    )(x, wt.astype(BF16), b.reshape(1, n), z, g.reshape(1, n),
      gb.reshape(1, n))


def _softmax_ctx(q, k, v, scale):
    s = jax.lax.dot_general(q.astype(BF16), k.astype(BF16),
                            (((1,), (1,)), ((), ())),
                            preferred_element_type=F32) * scale
    m = jnp.max(s, axis=-1, keepdims=True)
    e = jnp.exp(s - m)
    p = e / jnp.sum(e, axis=-1, keepdims=True)
    return _dot(p, v)


def _attn_full_body(q_ref, k_ref, v_ref, o_ref, scale, dh):
    q2 = q_ref[...]
    k2 = k_ref[...]
    v2 = v_ref[...]
    outs = []
    for i in range(2):
        sl = slice(i * dh, (i + 1) * dh)
        outs.append(_softmax_ctx(q2[:, sl], k2[:, sl], v2[:, sl], scale))
    o_ref[...] = jnp.concatenate(outs, axis=1).astype(o_ref.dtype)


def _attn_full(qkv, b, t, h, dh, d):
    scale = 1.0 / math.sqrt(float(dh))
    hp = h // 2
    return pl.pallas_call(
        functools.partial(_attn_full_body, scale=scale, dh=dh),
        grid=(b, hp),
        in_specs=[
            pl.BlockSpec((t, 2 * dh), lambda i, j: (i, j)),
            pl.BlockSpec((t, 2 * dh), lambda i, j, _hp=hp: (i, _hp + j)),
            pl.BlockSpec((t, 2 * dh), lambda i, j, _hp=hp: (i, 2 * _hp + j)),
        ],
        out_specs=pl.BlockSpec((t, 2 * dh), lambda i, j: (i, j)),
        out_shape=jax.ShapeDtypeStruct((b * t, d), BF16),
    )(qkv, qkv, qkv)


def _attn_last_body(zl_ref, wq_ref, bq_ref, k_ref, v_ref, o_ref, scale, dh):
    zl = zl_ref[0]
    q2 = _dot(zl, wq_ref[...]) + bq_ref[...]
    k2 = k_ref[...]
    v2 = v_ref[...]
    outs = []
    for i in range(2):
        sl = slice(i * dh, (i + 1) * dh)
        outs.append(_softmax_ctx(q2[:, sl], k2[:, sl], v2[:, sl], scale))
    o_ref[0] = jnp.concatenate(outs, axis=1)


def _attn_last(z_last3, wqt, bq, kv, b, t, h, dh, d):
    scale = 1.0 / math.sqrt(float(dh))
    hp = h // 2
    return pl.pallas_call(
        functools.partial(_attn_last_body, scale=scale, dh=dh),
        grid=(b, hp),
        in_specs=[
            pl.BlockSpec((1, 1, d), lambda i, j: (i, 0, 0)),
            pl.BlockSpec((d, 2 * dh), lambda i, j: (0, j)),
            pl.BlockSpec((1, 2 * dh), lambda i, j: (0, j)),
            pl.BlockSpec((t, 2 * dh), lambda i, j: (i, j)),
            pl.BlockSpec((t, 2 * dh), lambda i, j, _hp=hp: (i, _hp + j)),
        ],
        out_specs=pl.BlockSpec((1, 1, 2 * dh), lambda i, j: (i, 0, j)),
        out_shape=jax.ShapeDtypeStruct((b, 1, d), F32),
    )(z_last3, wqt.astype(BF16), bq.reshape(1, d), kv, kv)


def _tail_body(zl_ref, ctx_ref, wo_ref, bo_ref, g1_ref, b1n_ref,
               w1_ref, bf1_ref, w2_ref, bf2_ref, g2_ref, b2n_ref,
               fg_ref, fb_ref, gw_ref, gb_ref,
               ew1_ref, eb1_ref, ew2_ref, eb2_ref,
               mg_ref, mb_ref, wd_ref, bd_ref, xl_ref, o_ref, n_exp):
    zl = zl_ref[...]
    ctx = ctx_ref[...]
    z1 = _ln(zl + _dot(ctx, wo_ref[...]) + bo_ref[...],
             g1_ref[...], b1n_ref[...])
    ffh = _gelu(_dot(z1, w1_ref[...]) + bf1_ref[...])
    z2 = _ln(z1 + _dot(ffh, w2_ref[...]) + bf2_ref[...],
             g2_ref[...], b2n_ref[...])
    zf = _ln(z2, fg_ref[...], fb_ref[...])

    logits = jnp.dot(zf, gw_ref[...],
                     preferred_element_type=F32) + gb_ref[...]
    lcols = [logits[:, e:e + 1] for e in range(n_exp)]
    m1 = lcols[0]
    for le in lcols[1:]:
        m1 = jnp.maximum(m1, le)
    first, taken = [], None
    for le in lcols:
        is_e = (le == m1) if taken is None else jnp.logical_and(
            le == m1, jnp.logical_not(taken))
        first.append(is_e)
        taken = is_e if taken is None else jnp.logical_or(taken, is_e)
    masked = [jnp.where(f, -1e30, le) for f, le in zip(first, lcols)]
    m2 = masked[0]
    for le in masked[1:]:
        m2 = jnp.maximum(m2, le)
    second, taken2 = [], None
    for le in masked:
        is_e = (le == m2) if taken2 is None else jnp.logical_and(
            le == m2, jnp.logical_not(taken2))
        second.append(is_e)
        taken2 = is_e if taken2 is None else jnp.logical_or(taken2, is_e)
    w1c = 1.0 / (1.0 + jnp.exp(m2 - m1))
    mixed = jnp.zeros_like(zf)
    for e in range(n_exp):
        he = jnp.maximum(_dot(zf, ew1_ref[e]) + eb1_ref[e], 0.0)
        oe = _dot(he, ew2_ref[e]) + eb2_ref[e]
        coeff_e = w1c * first[e].astype(F32) + (1.0 - w1c) * second[e].astype(F32)
        mixed = mixed + coeff_e * oe

    zm = _ln(mixed + zf, mg_ref[...], mb_ref[...])
    xl = xl_ref[...]
    trend_last = (xl[:, 0] + xl[:, 1] + xl[:, 2] + xl[:, 3]) * (1.0 / 7.0)
    nr = wd_ref.shape[1]
    o_ref[...] = _dot(zm, wd_ref[...]) + bd_ref[...] + trend_last[:, :nr]


def kernel(x_series, Wp, bp, pos_enc, Wqkv, bqkv, Wo, bo, ln1_g, ln1_b,
           ln2_g, ln2_b, W1, b1, W2, b2, fin_g, fin_b, gate_W, gate_b,
           expW1, expb1, expW2, expb2, moe_g, moe_b, Wd, bd):
    b_, t_, f_ = x_series.shape
    d = Wp.shape[0]
    h = 8
    dh = d // h
    n_exp = gate_W.shape[0]
    nr = Wd.shape[0]
    n = b_ * t_

    z = pl.pallas_call(
        _decomp_proj_body,
        grid=(b_,),
        in_specs=[
            pl.BlockSpec((1, t_, f_), lambda i: (i, 0, 0)),
            pl.BlockSpec((f_, d), lambda i: (0, 0)),
            pl.BlockSpec((1, d), lambda i: (0, 0)),
            pl.BlockSpec((t_, d), lambda i: (0, 0)),
        ],
        out_specs=pl.BlockSpec((1, t_, d), lambda i: (i, 0, 0)),
        out_shape=jax.ShapeDtypeStruct((b_, t_, d), F32),
    )(x_series, Wp.T.astype(BF16), bp.reshape(1, d), pos_enc[:t_])
    z = z.reshape(n, d)

    qkv = _mm_bias(z, Wqkv[0].T, bqkv[0], out_dtype=BF16)
    ctx = _attn_full(qkv, b_, t_, h, dh, d)
    z = _proj_res_ln(ctx, Wo[0].T, bo[0], z, ln1_g[0], ln1_b[0])
    ffh = _mm_bias(z, W1[0].T, b1[0], act="gelu", out_dtype=BF16)
    z = _proj_res_ln(ffh, W2[0].T, b2[0], z, ln2_g[0], ln2_b[0])

    kv = _mm_bias(z, Wqkv[1, d:].T, bqkv[1, d:], out_dtype=BF16)
    z_last = z.reshape(b_, t_, d)[:, -1, :]
    ctx_last = _attn_last(z_last.reshape(b_, 1, d), Wqkv[1, :d].T,
                          bqkv[1, :d], kv, b_, t_, h, dh, d)
    ctx_last = ctx_last.reshape(b_, d)

    def fullspec(shape):
        nd = len(shape)
        return pl.BlockSpec(shape, lambda *a, _nd=nd: (0,) * _nd)

    x_last4 = x_series[:, t_ - 4:, :]
    args = (
        z_last, ctx_last, Wo[1].T.astype(BF16), bo[1].reshape(1, d),
        ln1_g[1].reshape(1, d), ln1_b[1].reshape(1, d),
        W1[1].T.astype(BF16), b1[1].reshape(1, 4 * d),
        W2[1].T.astype(BF16), b2[1].reshape(1, d),
        ln2_g[1].reshape(1, d), ln2_b[1].reshape(1, d),
        fin_g.reshape(1, d), fin_b.reshape(1, d),
        gate_W.T, gate_b.reshape(1, n_exp),
        expW1.transpose(0, 2, 1).astype(BF16),
        expb1.reshape(n_exp, 1, 2 * d),
        expW2.transpose(0, 2, 1).astype(BF16),
        expb2.reshape(n_exp, 1, d),
        moe_g.reshape(1, d), moe_b.reshape(1, d),
        Wd.T.astype(BF16), bd.reshape(1, nr), x_last4,
    )
    out = pl.pallas_call(
        functools.partial(_tail_body, n_exp=n_exp),
        in_specs=[fullspec(a.shape) for a in args],
        out_specs=fullspec((b_, nr)),
        out_shape=jax.ShapeDtypeStruct((b_, nr), F32),
        compiler_params=pltpu.CompilerParams(
            vmem_limit_bytes=60 * 1024 * 1024),
    )(*args)
    return out


# speedup vs baseline: 3.3425x; 1.3099x over previous
import functools
import math

import jax
import jax.numpy as jnp
from jax.experimental import pallas as pl
from jax.experimental.pallas import tpu as pltpu

F32 = jnp.float32
BF16 = jnp.bfloat16


def _ln(x, g, b):
    m = jnp.mean(x, axis=-1, keepdims=True)
    v = jnp.mean((x - m) ** 2, axis=-1, keepdims=True)
    return (x - m) * jax.lax.rsqrt(v + 1e-5) * g + b


def _gelu(x):
    return 0.5 * x * (1.0 + jax.lax.erf(x * (1.0 / math.sqrt(2.0))))


def _dot(a, b):
    return jnp.dot(a.astype(BF16), b.astype(BF16),
                   preferred_element_type=F32)


def _softmax_ctx(q, k, v, scale):
    s = jax.lax.dot_general(q.astype(BF16), k.astype(BF16),
                            (((1,), (1,)), ((), ())),
                            preferred_element_type=F32) * scale
    m = jnp.max(s, axis=-1, keepdims=True)
    e = jnp.exp(s - m)
    p = e / jnp.sum(e, axis=-1, keepdims=True)
    return _dot(p, v)


def _layer1_body(x_ref, wp_ref, bp_ref, pe_ref,
                 wqkv_ref, bqkv_ref, wo_ref, bo_ref, g1_ref, b1n_ref,
                 w1_ref, bf1_ref, w2_ref, bf2_ref, g2_ref, b2n_ref,
                 o_ref, qkv_s, ffh_s, h, dh, scale):
    x = x_ref[0]
    t_, f_ = x.shape
    acc = x
    for off in (1, 2, 3):
        zpad = jnp.zeros((off, f_), x.dtype)
        acc = acc + jnp.concatenate([zpad, x[: t_ - off]], axis=0)
        acc = acc + jnp.concatenate([x[off:], zpad], axis=0)
    trend = acc * (1.0 / 7.0)
    resid = x - trend
    z0 = _dot(resid, wp_ref[...]) + bp_ref[...] + pe_ref[...]

    d = z0.shape[1]
    qkv_s[...] = (_dot(z0, wqkv_ref[...]) + bqkv_ref[...]).astype(BF16)
    parts = []
    for hh in range(h):
        q = qkv_s[:, hh * dh:(hh + 1) * dh]
        k = qkv_s[:, d + hh * dh:d + (hh + 1) * dh]
        v = qkv_s[:, 2 * d + hh * dh:2 * d + (hh + 1) * dh]
        parts.append(_softmax_ctx(q, k, v, scale))
    ctx = jnp.concatenate(parts, axis=1)

    z1 = _ln(z0 + _dot(ctx, wo_ref[...]) + bo_ref[...],
             g1_ref[...], b1n_ref[...])
    ffh_s[...] = _gelu(_dot(z1, w1_ref[...]) + bf1_ref[...]).astype(BF16)
    z2 = _ln(z1 + _dot(ffh_s[...], w2_ref[...]) + bf2_ref[...],
             g2_ref[...], b2n_ref[...])
    o_ref[0] = z2


def _attn2_body(z_ref, wkv_ref, bkv_ref, wq_ref, bq_ref, o_ref, kv_s,
                h, dh, scale):
    zb = z_ref[0]
    t_, d = zb.shape
    kv_s[...] = (_dot(zb, wkv_ref[...]) + bkv_ref[...]).astype(BF16)
    zl = zb[t_ - 1:t_, :]
    q = _dot(zl, wq_ref[...]) + bq_ref[...]
    parts = []
    for hh in range(h):
        qh = q[:, hh * dh:(hh + 1) * dh]
        k = kv_s[:, hh * dh:(hh + 1) * dh]
        v = kv_s[:, d + hh * dh:d + (hh + 1) * dh]
        parts.append(_softmax_ctx(qh, k, v, scale))
    o_ref[0] = jnp.concatenate(parts, axis=1)


def _tail_body(zl_ref, ctx_ref, wo_ref, bo_ref, g1_ref, b1n_ref,
               w1_ref, bf1_ref, w2_ref, bf2_ref, g2_ref, b2n_ref,
               fg_ref, fb_ref, gw_ref, gb_ref,
               ew1_ref, eb1_ref, ew2_ref, eb2_ref,
               mg_ref, mb_ref, wd_ref, bd_ref, xl_ref, o_ref, n_exp):
    zl = zl_ref[...]
    ctx = ctx_ref[...]
    z1 = _ln(zl + _dot(ctx, wo_ref[...]) + bo_ref[...],
             g1_ref[...], b1n_ref[...])
    ffh = _gelu(_dot(z1, w1_ref[...]) + bf1_ref[...])
    z2 = _ln(z1 + _dot(ffh, w2_ref[...]) + bf2_ref[...],
             g2_ref[...], b2n_ref[...])
    zf = _ln(z2, fg_ref[...], fb_ref[...])

    logits = jnp.dot(zf, gw_ref[...],
                     preferred_element_type=F32) + gb_ref[...]
    lcols = [logits[:, e:e + 1] for e in range(n_exp)]
    m1 = lcols[0]
    for le in lcols[1:]:
        m1 = jnp.maximum(m1, le)
    first, taken = [], None
    for le in lcols:
        is_e = (le == m1) if taken is None else jnp.logical_and(
            le == m1, jnp.logical_not(taken))
        first.append(is_e)
        taken = is_e if taken is None else jnp.logical_or(taken, is_e)
    masked = [jnp.where(f, -1e30, le) for f, le in zip(first, lcols)]
    m2 = masked[0]
    for le in masked[1:]:
        m2 = jnp.maximum(m2, le)
    second, taken2 = [], None
    for le in masked:
        is_e = (le == m2) if taken2 is None else jnp.logical_and(
            le == m2, jnp.logical_not(taken2))
        second.append(is_e)
        taken2 = is_e if taken2 is None else jnp.logical_or(taken2, is_e)
    w1c = 1.0 / (1.0 + jnp.exp(m2 - m1))
    mixed = jnp.zeros_like(zf)
    for e in range(n_exp):
        he = jnp.maximum(_dot(zf, ew1_ref[e]) + eb1_ref[e], 0.0)
        oe = _dot(he, ew2_ref[e]) + eb2_ref[e]
        coeff_e = w1c * first[e].astype(F32) + (1.0 - w1c) * second[e].astype(F32)
        mixed = mixed + coeff_e * oe

    zm = _ln(mixed + zf, mg_ref[...], mb_ref[...])
    xl = xl_ref[...]
    trend_last = (xl[:, 0] + xl[:, 1] + xl[:, 2] + xl[:, 3]) * (1.0 / 7.0)
    nr = wd_ref.shape[1]
    o_ref[...] = _dot(zm, wd_ref[...]) + bd_ref[...] + trend_last[:, :nr]


def kernel(x_series, Wp, bp, pos_enc, Wqkv, bqkv, Wo, bo, ln1_g, ln1_b,
           ln2_g, ln2_b, W1, b1, W2, b2, fin_g, fin_b, gate_W, gate_b,
           expW1, expb1, expW2, expb2, moe_g, moe_b, Wd, bd):
    b_, t_, f_ = x_series.shape
    d = Wp.shape[0]
    h = 8
    dh = d // h
    n_exp = gate_W.shape[0]
    nr = Wd.shape[0]
    scale = 1.0 / math.sqrt(float(dh))

    def bcast2d(v):
        return v.reshape(1, v.shape[-1])

    row = lambda i: (0, 0)
    z = pl.pallas_call(
        functools.partial(_layer1_body, h=h, dh=dh, scale=scale),
        grid=(b_,),
        in_specs=[
            pl.BlockSpec((1, t_, f_), lambda i: (i, 0, 0)),
            pl.BlockSpec((f_, d), row),
            pl.BlockSpec((1, d), row),
            pl.BlockSpec((t_, d), row),
            pl.BlockSpec((d, 3 * d), row),
            pl.BlockSpec((1, 3 * d), row),
            pl.BlockSpec((d, d), row),
            pl.BlockSpec((1, d), row),
            pl.BlockSpec((1, d), row),
            pl.BlockSpec((1, d), row),
            pl.BlockSpec((d, 4 * d), row),
            pl.BlockSpec((1, 4 * d), row),
            pl.BlockSpec((4 * d, d), row),
            pl.BlockSpec((1, d), row),
            pl.BlockSpec((1, d), row),
            pl.BlockSpec((1, d), row),
        ],
        out_specs=pl.BlockSpec((1, t_, d), lambda i: (i, 0, 0)),
        out_shape=jax.ShapeDtypeStruct((b_, t_, d), F32),
        scratch_shapes=[
            pltpu.VMEM((t_, 3 * d), BF16),
            pltpu.VMEM((t_, 4 * d), BF16),
        ],
        compiler_params=pltpu.CompilerParams(
            vmem_limit_bytes=60 * 1024 * 1024),
    )(x_series, Wp.T.astype(BF16), bcast2d(bp), pos_enc[:t_],
      Wqkv[0].T.astype(BF16), bcast2d(bqkv[0]),
      Wo[0].T.astype(BF16), bcast2d(bo[0]),
      bcast2d(ln1_g[0]), bcast2d(ln1_b[0]),
      W1[0].T.astype(BF16), bcast2d(b1[0]),
      W2[0].T.astype(BF16), bcast2d(b2[0]),
      bcast2d(ln2_g[0]), bcast2d(ln2_b[0]))

    ctx_last = pl.pallas_call(
        functools.partial(_attn2_body, h=h, dh=dh, scale=scale),
        grid=(b_,),
        in_specs=[
            pl.BlockSpec((1, t_, d), lambda i: (i, 0, 0)),
            pl.BlockSpec((d, 2 * d), row),
            pl.BlockSpec((1, 2 * d), row),
            pl.BlockSpec((d, d), row),
            pl.BlockSpec((1, d), row),
        ],
        out_specs=pl.BlockSpec((1, 1, d), lambda i: (i, 0, 0)),
        out_shape=jax.ShapeDtypeStruct((b_, 1, d), F32),
        scratch_shapes=[pltpu.VMEM((t_, 2 * d), BF16)],
        compiler_params=pltpu.CompilerParams(
            vmem_limit_bytes=60 * 1024 * 1024),
    )(z, Wqkv[1, d:].T.astype(BF16), bcast2d(bqkv[1, d:]),
      Wqkv[1, :d].T.astype(BF16), bcast2d(bqkv[1, :d]))

    z_last = z[:, -1, :]
    ctx_last = ctx_last.reshape(b_, d)

    def fullspec(shape):
        nd = len(shape)
        return pl.BlockSpec(shape, lambda *a, _nd=nd: (0,) * _nd)

    x_last4 = x_series[:, t_ - 4:, :]
    args = (
        z_last, ctx_last, Wo[1].T.astype(BF16), bcast2d(bo[1]),
        bcast2d(ln1_g[1]), bcast2d(ln1_b[1]),
        W1[1].T.astype(BF16), bcast2d(b1[1]),
        W2[1].T.astype(BF16), bcast2d(b2[1]),
        bcast2d(ln2_g[1]), bcast2d(ln2_b[1]),
        bcast2d(fin_g), bcast2d(fin_b),
        gate_W.T, bcast2d(gate_b),
        expW1.transpose(0, 2, 1).astype(BF16),
        expb1.reshape(n_exp, 1, 2 * d),
        expW2.transpose(0, 2, 1).astype(BF16),
        expb2.reshape(n_exp, 1, d),
        bcast2d(moe_g), bcast2d(moe_b),
        Wd.T.astype(BF16), bcast2d(bd), x_last4,
    )
    out = pl.pallas_call(
        functools.partial(_tail_body, n_exp=n_exp),
        in_specs=[fullspec(a.shape) for a in args],
        out_specs=fullspec((b_, nr)),
        out_shape=jax.ShapeDtypeStruct((b_, nr), F32),
        compiler_params=pltpu.CompilerParams(
            vmem_limit_bytes=60 * 1024 * 1024),
    )(*args)
    return out


# speedup vs baseline: 4.3345x; 1.2968x over previous
import functools
import math

import jax
import jax.numpy as jnp
from jax.experimental import pallas as pl
from jax.experimental.pallas import tpu as pltpu

F32 = jnp.float32
BF16 = jnp.bfloat16
_DNUM = (((1,), (1,)), ((), ()))


def _ln(x, g, b):
    m = jnp.mean(x, axis=-1, keepdims=True)
    v = jnp.mean((x - m) ** 2, axis=-1, keepdims=True)
    return (x - m) * jax.lax.rsqrt(v + 1e-5) * g + b


def _gelu(x):
    return 0.5 * x * (1.0 + jax.lax.erf(x * (1.0 / math.sqrt(2.0))))


def _dotn(a, w):
    return jax.lax.dot_general(a.astype(BF16), w.astype(BF16), _DNUM,
                               preferred_element_type=F32)


def _softmax_ctx(q, k, v, scale):
    s = jax.lax.dot_general(q.astype(BF16), k, _DNUM,
                            preferred_element_type=F32) * scale
    m = jnp.max(s, axis=-1, keepdims=True)
    e = jnp.exp(s - m)
    p = e * jax.lax.reciprocal(jnp.sum(e, axis=-1, keepdims=True))
    return jax.lax.dot_general(p.astype(BF16), v, (((1,), (0,)), ((), ())),
                               preferred_element_type=F32)


def _body_a(x_ref, pe_ref, wp_ref, bp_ref,
            wqkv_ref, bqkv_ref, wo_ref, bo_ref, g1_ref, b1n_ref,
            w1_ref, bf1_ref, w2_ref, bf2_ref, g2_ref, b2n_ref,
            wkv_ref, bkv_ref, wq_ref, bq_ref,
            zl_ref, cl_ref,
            wp_s, wqkv_s, wo_s, w1_s, w2_s, wkv_s, wq_s, qkv_s, ffh_s, kv_s,
            h, dh, scale):
    wp_s[...] = wp_ref[...].astype(BF16)
    wqkv_s[...] = wqkv_ref[...].astype(BF16)
    wo_s[...] = wo_ref[...].astype(BF16)
    w1_s[...] = w1_ref[...].astype(BF16)
    w2_s[...] = w2_ref[...].astype(BF16)
    wkv_s[...] = wkv_ref[...].astype(BF16)
    wq_s[...] = wq_ref[...].astype(BF16)
    pe = pe_ref[...]
    b_ = x_ref.shape[0]
    d = wp_s.shape[0]
    zls, cls = [], []
    for b in range(b_):
        x = x_ref[b]
        t_, f_ = x.shape
        acc = x
        for off in (1, 2, 3):
            zpad = jnp.zeros((off, f_), x.dtype)
            acc = acc + jnp.concatenate([zpad, x[: t_ - off]], axis=0)
            acc = acc + jnp.concatenate([x[off:], zpad], axis=0)
        trend = acc * (1.0 / 7.0)
        resid = x - trend
        z0 = _dotn(resid, wp_s[...]) + bp_ref[...] + pe

        qkv_s[...] = (_dotn(z0, wqkv_s[...]) + bqkv_ref[...]).astype(BF16)
        parts = []
        for hh in range(h):
            q = qkv_s[:, hh * dh:(hh + 1) * dh]
            k = qkv_s[:, d + hh * dh:d + (hh + 1) * dh]
            v = qkv_s[:, 2 * d + hh * dh:2 * d + (hh + 1) * dh]
            parts.append(_softmax_ctx(q, k, v, scale))
        ctx = jnp.concatenate(parts, axis=1)

        z1 = _ln(z0 + _dotn(ctx, wo_s[...]) + bo_ref[...],
                 g1_ref[...], b1n_ref[...])
        ffh_s[...] = _gelu(_dotn(z1, w1_s[...]) + bf1_ref[...]).astype(BF16)
        z2 = _ln(z1 + _dotn(ffh_s[...], w2_s[...]) + bf2_ref[...],
                 g2_ref[...], b2n_ref[...])

        kv_s[...] = (_dotn(z2, wkv_s[...]) + bkv_ref[...]).astype(BF16)
        zl = z2[t_ - 1:t_, :]
        qlast = _dotn(zl, wq_s[...]) + bq_ref[...]
        parts = []
        for hh in range(h):
            qh = qlast[:, hh * dh:(hh + 1) * dh]
            k = kv_s[:, hh * dh:(hh + 1) * dh]
            v = kv_s[:, d + hh * dh:d + (hh + 1) * dh]
            parts.append(_softmax_ctx(qh, k, v, scale))
        zls.append(zl)
        cls.append(jnp.concatenate(parts, axis=1))
    zl_ref[...] = jnp.concatenate(zls, axis=0)
    cl_ref[...] = jnp.concatenate(cls, axis=0)


def _body_tail(zl_ref, ctx_ref, wo_ref, bo_ref, g1_ref, b1n_ref,
               w1_ref, bf1_ref, w2_ref, bf2_ref, g2_ref, b2n_ref,
               fg_ref, fb_ref, gw_ref, gb_ref,
               ew1_ref, eb1_ref, ew2_ref, eb2_ref,
               mg_ref, mb_ref, wd_ref, bd_ref, xl_ref, o_ref, n_exp):
    zl = zl_ref[...]
    ctx = ctx_ref[...]
    z1 = _ln(zl + _dotn(ctx, wo_ref[...]) + bo_ref[...],
             g1_ref[...], b1n_ref[...])
    ffh = _gelu(_dotn(z1, w1_ref[...]) + bf1_ref[...])
    z2 = _ln(z1 + _dotn(ffh, w2_ref[...]) + bf2_ref[...],
             g2_ref[...], b2n_ref[...])
    zf = _ln(z2, fg_ref[...], fb_ref[...])

    logits = jax.lax.dot_general(zf, gw_ref[...], _DNUM,
                                 preferred_element_type=F32) + gb_ref[...]
    lcols = [logits[:, e:e + 1] for e in range(n_exp)]
    m1 = lcols[0]
    for le in lcols[1:]:
        m1 = jnp.maximum(m1, le)
    first, taken = [], None
    for le in lcols:
        is_e = (le == m1) if taken is None else jnp.logical_and(
            le == m1, jnp.logical_not(taken))
        first.append(is_e)
        taken = is_e if taken is None else jnp.logical_or(taken, is_e)
    masked = [jnp.where(f, -1e30, le) for f, le in zip(first, lcols)]
    m2 = masked[0]
    for le in masked[1:]:
        m2 = jnp.maximum(m2, le)
    second, taken2 = [], None
    for le in masked:
        is_e = (le == m2) if taken2 is None else jnp.logical_and(
            le == m2, jnp.logical_not(taken2))
        second.append(is_e)
        taken2 = is_e if taken2 is None else jnp.logical_or(taken2, is_e)
    w1c = 1.0 / (1.0 + jnp.exp(m2 - m1))
    mixed = jnp.zeros_like(zf)
    for e in range(n_exp):
        he = jnp.maximum(_dotn(zf, ew1_ref[e]) + eb1_ref[e], 0.0)
        oe = _dotn(he, ew2_ref[e]) + eb2_ref[e]
        coeff_e = w1c * first[e].astype(F32) + (1.0 - w1c) * second[e].astype(F32)
        mixed = mixed + coeff_e * oe

    zm = _ln(mixed + zf, mg_ref[...], mb_ref[...])
    xl = xl_ref[...]
    trend_last = (xl[:, 0] + xl[:, 1] + xl[:, 2] + xl[:, 3]) * (1.0 / 7.0)
    nr = wd_ref.shape[0]
    o_ref[...] = _dotn(zm, wd_ref[...]) + bd_ref[...] + trend_last[:, :nr]


def kernel(x_series, Wp, bp, pos_enc, Wqkv, bqkv, Wo, bo, ln1_g, ln1_b,
           ln2_g, ln2_b, W1, b1, W2, b2, fin_g, fin_b, gate_W, gate_b,
           expW1, expb1, expW2, expb2, moe_g, moe_b, Wd, bd):
    b_, t_, f_ = x_series.shape
    d = Wp.shape[0]
    h = 8
    dh = d // h
    n_exp = gate_W.shape[0]
    nr = Wd.shape[0]
    scale = 1.0 / math.sqrt(float(dh))

    def bcast2d(v):
        return v.reshape(1, v.shape[-1])

    def fullspec(shape):
        nd = len(shape)
        return pl.BlockSpec(shape, lambda *a, _nd=nd: (0,) * _nd)

    args_a = (
        x_series, pos_enc[:t_], Wp, bcast2d(bp),
        Wqkv[0], bcast2d(bqkv[0]), Wo[0], bcast2d(bo[0]),
        bcast2d(ln1_g[0]), bcast2d(ln1_b[0]),
        W1[0], bcast2d(b1[0]), W2[0], bcast2d(b2[0]),
        bcast2d(ln2_g[0]), bcast2d(ln2_b[0]),
        Wqkv[1, d:], bcast2d(bqkv[1, d:]),
        Wqkv[1, :d], bcast2d(bqkv[1, :d]),
    )
    z_last, ctx_last = pl.pallas_call(
        functools.partial(_body_a, h=h, dh=dh, scale=scale),
        in_specs=[fullspec(a.shape) for a in args_a],
        out_specs=[fullspec((b_, d)), fullspec((b_, d))],
        out_shape=[jax.ShapeDtypeStruct((b_, d), F32),
                   jax.ShapeDtypeStruct((b_, d), F32)],
        scratch_shapes=[
            pltpu.VMEM((d, f_), BF16),
            pltpu.VMEM((3 * d, d), BF16),
            pltpu.VMEM((d, d), BF16),
            pltpu.VMEM((4 * d, d), BF16),
            pltpu.VMEM((d, 4 * d), BF16),
            pltpu.VMEM((2 * d, d), BF16),
            pltpu.VMEM((d, d), BF16),
            pltpu.VMEM((t_, 3 * d), BF16),
            pltpu.VMEM((t_, 4 * d), BF16),
            pltpu.VMEM((t_, 2 * d), BF16),
        ],
        compiler_params=pltpu.CompilerParams(
            vmem_limit_bytes=60 * 1024 * 1024),
    )(*args_a)

    x_last4 = x_series[:, t_ - 4:, :]
    args_t = (
        z_last, ctx_last, Wo[1], bcast2d(bo[1]),
        bcast2d(ln1_g[1]), bcast2d(ln1_b[1]),
        W1[1], bcast2d(b1[1]), W2[1], bcast2d(b2[1]),
        bcast2d(ln2_g[1]), bcast2d(ln2_b[1]),
        bcast2d(fin_g), bcast2d(fin_b),
        gate_W, bcast2d(gate_b),
        expW1, expb1.reshape(n_exp, 1, 2 * d),
        expW2, expb2.reshape(n_exp, 1, d),
        bcast2d(moe_g), bcast2d(moe_b),
        Wd, bcast2d(bd), x_last4,
    )
    out = pl.pallas_call(
        functools.partial(_body_tail, n_exp=n_exp),
        in_specs=[fullspec(a.shape) for a in args_t],
        out_specs=fullspec((b_, nr)),
        out_shape=jax.ShapeDtypeStruct((b_, nr), F32),
        compiler_params=pltpu.CompilerParams(
            vmem_limit_bytes=60 * 1024 * 1024),
    )(*args_t)
    return out


# speedup vs baseline: 4.3475x; 1.0030x over previous
import functools
import math

import jax
import jax.numpy as jnp
from jax.experimental import pallas as pl
from jax.experimental.pallas import tpu as pltpu

F32 = jnp.float32
BF16 = jnp.bfloat16
_DNUM = (((1,), (1,)), ((), ()))


def _ln(x, g, b):
    m = jnp.mean(x, axis=-1, keepdims=True)
    v = jnp.mean((x - m) ** 2, axis=-1, keepdims=True)
    return (x - m) * jax.lax.rsqrt(v + 1e-5) * g + b


def _gelu(x):
    return 0.5 * x * (1.0 + jax.lax.erf(x * (1.0 / math.sqrt(2.0))))


def _dotn(a, w):
    return jax.lax.dot_general(a.astype(BF16), w.astype(BF16), _DNUM,
                               preferred_element_type=F32)


def _softmax_ctx(q, k, v, scale):
    qs = (q * scale).astype(BF16)
    s = jax.lax.dot_general(qs, k, _DNUM, preferred_element_type=F32)
    m = jnp.max(s, axis=-1, keepdims=True)
    e = jnp.exp((s - m).astype(BF16))
    r = jax.lax.reciprocal(jnp.sum(e.astype(F32), axis=-1, keepdims=True))
    p = e * r.astype(BF16)
    return jax.lax.dot_general(p, v, (((1,), (0,)), ((), ())),
                               preferred_element_type=F32)


def _body_a(x_ref, pe_ref, wp_ref, bp_ref,
            wqkv_ref, bqkv_ref, wo_ref, bo_ref, g1_ref, b1n_ref,
            w1_ref, bf1_ref, w2_ref, bf2_ref, g2_ref, b2n_ref,
            wkv_ref, bkv_ref, wq_ref, bq_ref,
            zl_ref, cl_ref,
            wp_s, wqkv_s, wo_s, w1_s, w2_s, wkv_s, wq_s, qkv_s, ffh_s, kv_s,
            h, dh, scale):
    wp_s[...] = wp_ref[...].astype(BF16)
    wqkv_s[...] = wqkv_ref[...].astype(BF16)
    wo_s[...] = wo_ref[...].astype(BF16)
    w1_s[...] = w1_ref[...].astype(BF16)
    w2_s[...] = w2_ref[...].astype(BF16)
    wkv_s[...] = wkv_ref[...].astype(BF16)
    wq_s[...] = wq_ref[...].astype(BF16)
    pe = pe_ref[...]
    b_ = x_ref.shape[0]
    d = wp_s.shape[0]
    zls, cls = [], []
    for b in range(b_):
        x = x_ref[b]
        t_, f_ = x.shape
        acc = x
        for off in (1, 2, 3):
            zpad = jnp.zeros((off, f_), x.dtype)
            acc = acc + jnp.concatenate([zpad, x[: t_ - off]], axis=0)
            acc = acc + jnp.concatenate([x[off:], zpad], axis=0)
        trend = acc * (1.0 / 7.0)
        resid = x - trend
        z0 = _dotn(resid, wp_s[...]) + bp_ref[...] + pe

        qkv_s[...] = (_dotn(z0, wqkv_s[...]) + bqkv_ref[...]).astype(BF16)
        parts = []
        for hh in range(h):
            q = qkv_s[:, hh * dh:(hh + 1) * dh]
            k = qkv_s[:, d + hh * dh:d + (hh + 1) * dh]
            v = qkv_s[:, 2 * d + hh * dh:2 * d + (hh + 1) * dh]
            parts.append(_softmax_ctx(q, k, v, scale))
        ctx = jnp.concatenate(parts, axis=1)

        z1 = _ln(z0 + _dotn(ctx, wo_s[...]) + bo_ref[...],
                 g1_ref[...], b1n_ref[...])
        ffh_s[...] = _gelu(
            (_dotn(z1, w1_s[...]) + bf1_ref[...]).astype(BF16))
        z2 = _ln(z1 + _dotn(ffh_s[...], w2_s[...]) + bf2_ref[...],
                 g2_ref[...], b2n_ref[...])

        kv_s[...] = (_dotn(z2, wkv_s[...]) + bkv_ref[...]).astype(BF16)
        zl = z2[t_ - 1:t_, :]
        qlast = _dotn(zl, wq_s[...]) + bq_ref[...]
        parts = []
        for hh in range(h):
            qh = qlast[:, hh * dh:(hh + 1) * dh]
            k = kv_s[:, hh * dh:(hh + 1) * dh]
            v = kv_s[:, d + hh * dh:d + (hh + 1) * dh]
            parts.append(_softmax_ctx(qh, k, v, scale))
        zls.append(zl)
        cls.append(jnp.concatenate(parts, axis=1))
    zl_ref[...] = jnp.concatenate(zls, axis=0)
    cl_ref[...] = jnp.concatenate(cls, axis=0)


def _body_tail(zl_ref, ctx_ref, wo_ref, bo_ref, g1_ref, b1n_ref,
               w1_ref, bf1_ref, w2_ref, bf2_ref, g2_ref, b2n_ref,
               fg_ref, fb_ref, gw_ref, gb_ref,
               ew1_ref, eb1_ref, ew2_ref, eb2_ref,
               mg_ref, mb_ref, wd_ref, bd_ref, xl_ref, o_ref, n_exp):
    zl = zl_ref[...]
    ctx = ctx_ref[...]
    z1 = _ln(zl + _dotn(ctx, wo_ref[...]) + bo_ref[...],
             g1_ref[...], b1n_ref[...])
    ffh = _gelu(_dotn(z1, w1_ref[...]) + bf1_ref[...])
    z2 = _ln(z1 + _dotn(ffh, w2_ref[...]) + bf2_ref[...],
             g2_ref[...], b2n_ref[...])
    zf = _ln(z2, fg_ref[...], fb_ref[...])

    logits = jax.lax.dot_general(zf, gw_ref[...], _DNUM,
                                 preferred_element_type=F32) + gb_ref[...]
    lcols = [logits[:, e:e + 1] for e in range(n_exp)]
    m1 = lcols[0]
    for le in lcols[1:]:
        m1 = jnp.maximum(m1, le)
    first, taken = [], None
    for le in lcols:
        is_e = (le == m1) if taken is None else jnp.logical_and(
            le == m1, jnp.logical_not(taken))
        first.append(is_e)
        taken = is_e if taken is None else jnp.logical_or(taken, is_e)
    masked = [jnp.where(f, -1e30, le) for f, le in zip(first, lcols)]
    m2 = masked[0]
    for le in masked[1:]:
        m2 = jnp.maximum(m2, le)
    second, taken2 = [], None
    for le in masked:
        is_e = (le == m2) if taken2 is None else jnp.logical_and(
            le == m2, jnp.logical_not(taken2))
        second.append(is_e)
        taken2 = is_e if taken2 is None else jnp.logical_or(taken2, is_e)
    w1c = 1.0 / (1.0 + jnp.exp(m2 - m1))
    mixed = jnp.zeros_like(zf)
    for e in range(n_exp):
        he = jnp.maximum(_dotn(zf, ew1_ref[e]) + eb1_ref[e], 0.0)
        oe = _dotn(he, ew2_ref[e]) + eb2_ref[e]
        coeff_e = w1c * first[e].astype(F32) + (1.0 - w1c) * second[e].astype(F32)
        mixed = mixed + coeff_e * oe

    zm = _ln(mixed + zf, mg_ref[...], mb_ref[...])
    xl = xl_ref[...]
    trend_last = (xl[:, 0] + xl[:, 1] + xl[:, 2] + xl[:, 3]) * (1.0 / 7.0)
    nr = wd_ref.shape[0]
    o_ref[...] = _dotn(zm, wd_ref[...]) + bd_ref[...] + trend_last[:, :nr]


def kernel(x_series, Wp, bp, pos_enc, Wqkv, bqkv, Wo, bo, ln1_g, ln1_b,
           ln2_g, ln2_b, W1, b1, W2, b2, fin_g, fin_b, gate_W, gate_b,
           expW1, expb1, expW2, expb2, moe_g, moe_b, Wd, bd):
    b_, t_, f_ = x_series.shape
    d = Wp.shape[0]
    h = 8
    dh = d // h
    n_exp = gate_W.shape[0]
    nr = Wd.shape[0]
    scale = 1.0 / math.sqrt(float(dh))

    def bcast2d(v):
        return v.reshape(1, v.shape[-1])

    def fullspec(shape):
        nd = len(shape)
        return pl.BlockSpec(shape, lambda *a, _nd=nd: (0,) * _nd)

    args_a = (
        x_series, pos_enc[:t_], Wp, bcast2d(bp),
        Wqkv[0], bcast2d(bqkv[0]), Wo[0], bcast2d(bo[0]),
        bcast2d(ln1_g[0]), bcast2d(ln1_b[0]),
        W1[0], bcast2d(b1[0]), W2[0], bcast2d(b2[0]),
        bcast2d(ln2_g[0]), bcast2d(ln2_b[0]),
        Wqkv[1, d:], bcast2d(bqkv[1, d:]),
        Wqkv[1, :d], bcast2d(bqkv[1, :d]),
    )
    z_last, ctx_last = pl.pallas_call(
        functools.partial(_body_a, h=h, dh=dh, scale=scale),
        in_specs=[fullspec(a.shape) for a in args_a],
        out_specs=[fullspec((b_, d)), fullspec((b_, d))],
        out_shape=[jax.ShapeDtypeStruct((b_, d), F32),
                   jax.ShapeDtypeStruct((b_, d), F32)],
        scratch_shapes=[
            pltpu.VMEM((d, f_), BF16),
            pltpu.VMEM((3 * d, d), BF16),
            pltpu.VMEM((d, d), BF16),
            pltpu.VMEM((4 * d, d), BF16),
            pltpu.VMEM((d, 4 * d), BF16),
            pltpu.VMEM((2 * d, d), BF16),
            pltpu.VMEM((d, d), BF16),
            pltpu.VMEM((t_, 3 * d), BF16),
            pltpu.VMEM((t_, 4 * d), BF16),
            pltpu.VMEM((t_, 2 * d), BF16),
        ],
        compiler_params=pltpu.CompilerParams(
            vmem_limit_bytes=60 * 1024 * 1024),
    )(*args_a)

    x_last4 = x_series[:, t_ - 4:, :]
    args_t = (
        z_last, ctx_last, Wo[1], bcast2d(bo[1]),
        bcast2d(ln1_g[1]), bcast2d(ln1_b[1]),
        W1[1], bcast2d(b1[1]), W2[1], bcast2d(b2[1]),
        bcast2d(ln2_g[1]), bcast2d(ln2_b[1]),
        bcast2d(fin_g), bcast2d(fin_b),
        gate_W, bcast2d(gate_b),
        expW1, expb1.reshape(n_exp, 1, 2 * d),
        expW2, expb2.reshape(n_exp, 1, d),
        bcast2d(moe_g), bcast2d(moe_b),
        Wd, bcast2d(bd), x_last4,
    )
    out = pl.pallas_call(
        functools.partial(_body_tail, n_exp=n_exp),
        in_specs=[fullspec(a.shape) for a in args_t],
        out_specs=fullspec((b_, nr)),
        out_shape=jax.ShapeDtypeStruct((b_, nr), F32),
        compiler_params=pltpu.CompilerParams(
            vmem_limit_bytes=60 * 1024 * 1024),
    )(*args_t)
    return out
